# baseline (speedup 1.0000x reference)
import functools

import jax
import jax.numpy as jnp
from jax import lax
from jax.experimental import pallas as pl
from jax.experimental.pallas import tpu as pltpu
from jax.experimental.pallas import tpu_sc as plsc

LANE = 128
D = 32
CP = 129
NBUF = 4
M = 3


def _make_gather(n_vocab: int, b0: int, b1: int):
    info = plsc.get_sparse_core_info()
    nw = info.num_cores * info.num_subcores
    bw = b0 // nw
    nq = bw // LANE
    assert nq == NBUF and b0 % (LANE * nw) == 0 and D % 8 == 0
    ntc = b0 // LANE
    ntr = (b1 + 7) // 8
    mesh = plsc.VectorSubcoreMesh(core_axis_name="c", subcore_axis_name="s")

    @functools.partial(
        pl.kernel,
        out_type=jax.ShapeDtypeStruct((b1, D // 8, ntc, 8, LANE), jnp.float32),
        mesh=mesh,
        scratch_types=[
            pltpu.VMEM((ntr, nq, 8, LANE), jnp.int32),
            pltpu.VMEM((NBUF, LANE, D), jnp.float32),
            pltpu.VMEM((NBUF, D, CP), jnp.float32),
        ]
        + [pltpu.SemaphoreType.DMA] * (2 * NBUF),
        compiler_params=pltpu.CompilerParams(
            use_tc_tiling_on_sc=False, needs_layout_passes=False
        ),
    )
    def k(idxt_hbm, table_hbm, out_hbm, idx_v, rows_v, t_v, *sems):
        gsem = sems[:NBUF]
        ssem = sems[NBUF:]
        wid = lax.axis_index("s") * info.num_cores + lax.axis_index("c")
        pltpu.sync_copy(idxt_hbm.at[:, pl.ds(nq * wid, nq), :, :], idx_v)

        dvec = [lax.iota(jnp.int32, 16) + 16 * g for g in range(D // 16)]

        def ilist(j, q):
            return idx_v.at[j // 8, q, j % 8, :]

        def g_start(j, q):
            pltpu.async_copy(
                table_hbm.at[ilist(j, q)], rows_v.at[q], gsem[q],
            )

        def g_wait(j, q):
            pltpu.make_async_copy(
                table_hbm.at[ilist(j, q)], rows_v.at[q], gsem[q],
            ).wait()

        def s_start(j, q):
            for tr in range(D // 8):
                pltpu.async_copy(
                    t_v.at[q, pl.ds(8 * tr, 8), pl.ds(0, LANE)],
                    out_hbm.at[j, tr, nq * wid + q], ssem[q],
                )

        def s_wait(j, q):
            for tr in range(D // 8):
                pltpu.make_async_copy(
                    t_v.at[q, pl.ds(8 * tr, 8), pl.ds(0, LANE)],
                    out_hbm.at[j, tr, nq * wid + q], ssem[q],
                ).wait()

        def transpose(q):
            for c in range(LANE):
                cv = jnp.full((16,), c, jnp.int32)
                for g in range(D // 16):
                    v = rows_v[q, c, pl.ds(16 * g, 16)]
                    plsc.store_scatter(t_v.at[q], [dvec[g], cv], v)

        def step(j, q, wait_store, prefetch):
            g_wait(j, q)
            if wait_store:
                s_wait(j - 1, q)
            transpose(q)
            s_start(j, q)
            if prefetch:
                q2 = (q + M) % nq
                g_start(j + (q + M) // nq, q2)

        for q in range(M):
            g_start(0, q)
        for q in range(nq):
            step(0, q, wait_store=False, prefetch=True)

        @pl.loop(1, b1 - 1)
        def _(j):
            for q in range(nq):
                step(j, q, wait_store=True, prefetch=True)

        for q in range(nq):
            step(b1 - 1, q, wait_store=True, prefetch=(q + M < nq))
        for q in range(nq):
            s_wait(b1 - 1, q)

    return k


def kernel(idx, table):
    b0, b1 = idx.shape
    n_vocab, d = table.shape
    ntr = (b1 + 7) // 8
    idxt = jnp.transpose(idx.astype(jnp.int32), (1, 0))
    idxp = jnp.pad(idxt, ((0, 8 * ntr - b1), (0, 0)))
    idxp = lax.optimization_barrier(idxp)
    idx4 = jnp.transpose(
        idxp.reshape(ntr, 8, b0 // LANE, LANE), (0, 2, 1, 3)
    )
    w = _make_gather(n_vocab, b0, b1)(idx4, table)
    return jnp.transpose(w, (2, 4, 0, 1, 3)).reshape(b0, b1, d)

def _probe():
    import jax as _jax, jax.numpy as _jnp
    i = _jax.ShapeDtypeStruct((16384, 50), _jnp.int32)
    t = _jax.ShapeDtypeStruct((1000000, 32), _jnp.float32)
    txt = _jax.jit(kernel).lower(i, t).compile().as_text()
    import re as _re
    for l in txt.splitlines():
        s = l.strip()
        if _re.match(r"(ROOT )?%(reshape|pad|copy|bitcast|fusion|transpose|convert)", s):
            print("HLO:", s[:240])
    for l in txt.splitlines():
        if "entry_computation_layout" in l:
            print("HLO:", l.strip()[:400])
_probe()
